# baseline (speedup 1.0000x reference)
import functools

import jax
import jax.numpy as jnp
from jax import lax
from jax.experimental import pallas as pl
from jax.experimental.pallas import tpu as pltpu
from jax.experimental.pallas import tpu_sc as plsc

VOCAB = 1000001
EMBED_DIM = 32
BATCH = 16384

_NC = 2
_NS = 16
_NW = _NC * _NS
_B_PER_W = BATCH // _NW
_CHUNK = 128
_NCHUNK = _B_PER_W // _CHUNK

_mesh = plsc.VectorSubcoreMesh(core_axis_name="c", subcore_axis_name="s")


@functools.partial(
    pl.kernel,
    mesh=_mesh,
    out_type=jax.ShapeDtypeStruct((BATCH, EMBED_DIM), jnp.float32),
    scratch_types=[
        pltpu.VMEM((_NCHUNK, _CHUNK), jnp.int32),
        pltpu.VMEM((_B_PER_W, EMBED_DIM), jnp.float32),
        pltpu.SemaphoreType.DMA,
    ],
    compiler_params=pltpu.CompilerParams(use_tc_tiling_on_sc=False),
)
def _gather_kernel(idx_hbm, table_hbm, out_hbm, idx_v, rows_v, sem):
    wid = lax.axis_index("s") * _NC + lax.axis_index("c")
    base = wid * _B_PER_W
    pltpu.sync_copy(idx_hbm.at[wid], idx_v)
    copies = []
    for c in range(_NCHUNK):
        copies.append(
            pltpu.async_copy(
                table_hbm.at[idx_v.at[c]],
                rows_v.at[pl.ds(c * _CHUNK, _CHUNK)],
                sem,
            )
        )
    for cp in copies:
        cp.wait()
    pltpu.sync_copy(rows_v, out_hbm.at[pl.ds(base, _B_PER_W)])


def kernel(inputs, table):
    idx = inputs.astype(jnp.int32).reshape(_NW, _NCHUNK, _CHUNK)
    return _gather_kernel(idx, table)


# speedup vs baseline: 3.7376x; 3.7376x over previous
import functools

import jax
import jax.numpy as jnp
from jax import lax
from jax.experimental import pallas as pl
from jax.experimental.pallas import tpu as pltpu
from jax.experimental.pallas import tpu_sc as plsc

VOCAB = 1000001
EMBED_DIM = 32
BATCH = 16384

_NC = 2
_NS = 16
_NW = _NC * _NS
_B_PER_W = BATCH // _NW
_G = 8
_NGROUP = _B_PER_W // _G

_mesh = plsc.VectorSubcoreMesh(core_axis_name="c", subcore_axis_name="s")


@functools.partial(
    pl.kernel,
    mesh=_mesh,
    out_type=jax.ShapeDtypeStruct((EMBED_DIM, BATCH), jnp.float32),
    scratch_types=[
        pltpu.VMEM((_B_PER_W + 16,), jnp.int32),
    ] + [
        pltpu.VMEM((EMBED_DIM, 128), jnp.float32)
        for _ in range(2 * _G)
    ] + [
        pltpu.VMEM((EMBED_DIM, _B_PER_W), jnp.float32),
        pltpu.SemaphoreType.DMA,
        pltpu.SemaphoreType.DMA,
    ],
    compiler_params=pltpu.CompilerParams(needs_layout_passes=False),
)
def _colgather_kernel(idx_hbm, table_t_hbm, out_t_hbm,
                      idx_v, *rest):
    slots = [list(rest[0:_G]), list(rest[_G:2 * _G])]
    out_block, sem0, sem1 = rest[2 * _G], rest[2 * _G + 1], rest[2 * _G + 2]
    wid = lax.axis_index("s") * _NC + lax.axis_index("c")
    base = wid * _B_PER_W
    pltpu.sync_copy(idx_hbm.at[pl.ds(base, _B_PER_W)], idx_v.at[pl.ds(0, _B_PER_W)])
    idx_v[pl.ds(_B_PER_W, 16)] = jnp.zeros((16,), jnp.int32)

    lanes16 = lax.iota(jnp.int32, 16)

    def fire(g, bank, sem):
        gb = g * _G
        cvec = idx_v[pl.ds(gb, 16)]
        for i in range(_G):
            c = cvec[i]
            off = lax.shift_left(lax.shift_right_logical(c, 7), 7)
            off = pl.multiple_of(off, 128)
            pltpu.async_copy(
                table_t_hbm.at[:, pl.ds(off, 128)],
                slots[bank][i],
                sem,
            )

    def drain(bank, sem):
        for i in range(_G):
            pltpu.make_async_copy(
                table_t_hbm.at[:, pl.ds(0, 128)],
                slots[bank][i],
                sem,
            ).wait()

    def extract(g, bank):
        gb = g * _G
        for i in range(_G):
            c_b = plsc.load_gather(idx_v, [jnp.full((16,), gb + i, jnp.int32)])
            lane_b = lax.bitwise_and(c_b, jnp.int32(127))
            col_b = jnp.full((16,), gb + i, jnp.int32)
            lo = plsc.load_gather(slots[bank][i], [lanes16, lane_b])
            hi = plsc.load_gather(slots[bank][i], [lanes16 + 16, lane_b])
            plsc.store_scatter(out_block, [lanes16, col_b], lo)
            plsc.store_scatter(out_block, [lanes16 + 16, col_b], hi)

    fire(0, 0, sem0)

    def body(j, carry):
        g0 = 2 * j
        fire(g0 + 1, 1, sem1)
        drain(0, sem0)
        extract(g0, 0)
        g_next = lax.rem(g0 + 2, _NGROUP)
        fire(g_next, 0, sem0)
        drain(1, sem1)
        extract(g0 + 1, 1)
        return carry

    lax.fori_loop(0, _NGROUP // 2, body, 0)
    drain(0, sem0)

    pltpu.sync_copy(out_block, out_t_hbm.at[:, pl.ds(base, _B_PER_W)])


def kernel(inputs, table):
    idx = inputs.astype(jnp.int32)
    out_t = _colgather_kernel(idx, table.T)
    return out_t.T
